# Initial kernel scaffold; baseline (speedup 1.0000x reference)
import jax
import jax.numpy as jnp
from jax.experimental import pallas as pl


def kernel(x, edge_index, W1, b1, W2, b2):
    raise NotImplementedError("write your pallas kernel here")


# speedup vs baseline: 13.5155x; 13.5155x over previous
import functools

import jax
import jax.numpy as jnp
from jax import lax
from jax.experimental import pallas as pl
from jax.experimental.pallas import tpu as pltpu
from jax.experimental.pallas import tpu_sc as plsc

N_NODES = 10000
N_EDGES = 320000
D = 128

NC = 2
NS = 16
NW = NC * NS
NP = 10240
CH = 128
NCH = 79
EPW = CH * NCH
EPAD = NW * EPW
RPW = NP // NS

_mesh = plsc.VectorSubcoreMesh(core_axis_name="c", subcore_axis_name="s")


@functools.partial(
    pl.kernel,
    out_type=jax.ShapeDtypeStruct((NC, NP), jnp.float32),
    mesh=_mesh,
    scratch_types=[
        pltpu.VMEM((NCH, CH), jnp.int32),
        pltpu.VMEM((CH,), jnp.float32),
        pltpu.VMEM((RPW,), jnp.float32),
        pltpu.VMEM_SHARED((NP,), jnp.float32),
    ],
)
def _deg_kernel(dst_hbm, ones_hbm, zvec_hbm, out_hbm, idx_v, ones_v, stage_v, acc_sh):
    cid = lax.axis_index("c")
    sid = lax.axis_index("s")
    wid = cid * NS + sid
    pltpu.sync_copy(zvec_hbm, stage_v)
    pltpu.sync_copy(ones_hbm, ones_v)
    pltpu.sync_copy(stage_v, acc_sh.at[pl.ds(sid * RPW, RPW)])
    plsc.subcore_barrier()
    pltpu.sync_copy(dst_hbm.at[wid], idx_v)

    def body(j, carry):
        pltpu.sync_copy(ones_v, acc_sh.at[idx_v.at[j]], add=True)
        return carry

    lax.fori_loop(0, NCH, body, 0)
    plsc.subcore_barrier()
    pltpu.sync_copy(acc_sh.at[pl.ds(sid * RPW, RPW)], stage_v)
    pltpu.sync_copy(stage_v, out_hbm.at[cid, pl.ds(sid * RPW, RPW)])


@functools.partial(
    pl.kernel,
    out_type=jax.ShapeDtypeStruct((NC, NP, D), jnp.float32),
    mesh=_mesh,
    scratch_types=[
        pltpu.VMEM((NCH, CH), jnp.int32),
        pltpu.VMEM((NCH, CH), jnp.int32),
        pltpu.VMEM((CH, D), jnp.float32),
        pltpu.VMEM_SHARED((NP, D), jnp.float32),
        pltpu.SemaphoreType.DMA,
    ],
)
def _msg_kernel(y_hbm, src_hbm, dst_hbm, zrows_hbm, out_hbm,
                src_v, dst_v, rows_v, acc_sh, sem):
    cid = lax.axis_index("c")
    sid = lax.axis_index("s")
    wid = cid * NS + sid
    pltpu.sync_copy(zrows_hbm, rows_v)
    for r in range(RPW // CH):
        pltpu.sync_copy(rows_v, acc_sh.at[pl.ds(sid * RPW + r * CH, CH)])
    plsc.subcore_barrier()
    pltpu.sync_copy(src_hbm.at[wid], src_v)
    pltpu.sync_copy(dst_hbm.at[wid], dst_v)

    def body(j, carry):
        pltpu.async_copy(y_hbm.at[src_v.at[j]], rows_v, sem).wait()
        pltpu.sync_copy(rows_v, acc_sh.at[dst_v.at[j]], add=True)
        return carry

    lax.fori_loop(0, NCH, body, 0)
    plsc.subcore_barrier()
    for r in range(RPW // CH):
        base = sid * RPW + r * CH
        pltpu.sync_copy(acc_sh.at[pl.ds(base, CH)], rows_v)
        pltpu.sync_copy(rows_v, out_hbm.at[cid, pl.ds(base, CH)])


def _stage1_body(x_ref, w1_ref, degp_ref, y_ref, dinv_ref):
    degp = degp_ref[...]
    deg = degp[0, :N_NODES] + degp[1, :N_NODES] + 1.0
    dcol = lax.rsqrt(deg)[:, None]
    dinv_ref[...] = dcol
    xw = jnp.dot(x_ref[...], w1_ref[...], preferred_element_type=jnp.float32)
    y_ref[...] = xw * dcol


def _stage2_body(sp_ref, y_ref, dinv_ref, b1_ref, w2_ref, y2_ref):
    sp = sp_ref[...]
    s = sp[0, :N_NODES] + sp[1, :N_NODES]
    dcol = dinv_ref[...]
    h1 = jnp.maximum((s + y_ref[...]) * dcol + b1_ref[...], 0.0)
    y2_ref[...] = jnp.dot(h1, w2_ref[...], preferred_element_type=jnp.float32) * dcol


def _stage3_body(sp_ref, y2_ref, dinv_ref, b2_ref, out_ref):
    sp = sp_ref[...]
    s = sp[0, :N_NODES] + sp[1, :N_NODES]
    f = 2.0 * ((s + y2_ref[...]) * dinv_ref[...] + b2_ref[...])
    m = jnp.max(f, axis=1, keepdims=True)
    lse = jnp.log(jnp.sum(jnp.exp(f - m), axis=1, keepdims=True)) + m
    out_ref[...] = f - lse


_stage1 = pl.pallas_call(
    _stage1_body,
    out_shape=(
        jax.ShapeDtypeStruct((N_NODES, D), jnp.float32),
        jax.ShapeDtypeStruct((N_NODES, 1), jnp.float32),
    ),
)

_stage2 = pl.pallas_call(
    _stage2_body,
    out_shape=jax.ShapeDtypeStruct((N_NODES, D), jnp.float32),
)

_stage3 = pl.pallas_call(
    _stage3_body,
    out_shape=jax.ShapeDtypeStruct((N_NODES, D), jnp.float32),
)


def kernel(x, edge_index, W1, b1, W2, b2):
    ei = edge_index.astype(jnp.int32)
    npad = EPAD - N_EDGES
    src = jnp.concatenate([ei[0], jnp.zeros((npad,), jnp.int32)])
    dst = jnp.concatenate([ei[1], jnp.full((npad,), N_NODES, jnp.int32)])
    src = src.reshape(NW, NCH, CH)
    dst = dst.reshape(NW, NCH, CH)

    ones_ch = jnp.ones((CH,), jnp.float32)
    zvec = jnp.zeros((RPW,), jnp.float32)
    zrows = jnp.zeros((CH, D), jnp.float32)

    degp = _deg_kernel(dst, ones_ch, zvec)
    y, dinv = _stage1(x, W1, degp)
    s1p = _msg_kernel(y, src, dst, zrows)
    y2 = _stage2(s1p, y, dinv, b1, W2)
    s2p = _msg_kernel(y2, src, dst, zrows)
    return _stage3(s2p, y2, dinv, b2)
